# baseline (speedup 1.0000x reference)
import functools

import jax
import jax.numpy as jnp
from jax import lax
from jax.experimental import pallas as pl
from jax.experimental.pallas import tpu as pltpu
from jax.experimental.pallas import tpu_sc as plsc

B = 4096
F = 26
C = 1000
OUT_W = F * C
NC, NS, L = 2, 16, 16
NW = NC * NS
ROWS = B // NW
ZCHUNK = 5


def _sc_one_hot_body(x_hbm, out_hbm, x_v, buf0, buf1, buf2, buf3,
                     sem0, sem1, sem2, sem3):
    wid = lax.axis_index("s") * NC + lax.axis_index("c")
    base = wid * ROWS

    pltpu.sync_copy(x_hbm.at[pl.ds(base * F, ROWS * F)], x_v)

    lanes = lax.iota(jnp.int32, L)
    ones = jnp.full((L,), 1.0, jnp.float32)
    zeros = jnp.zeros((L,), jnp.float32)

    f0 = lanes
    f1 = jnp.minimum(lanes + 16, F - 1)
    m1 = lanes < (F - 16)

    def row_targets(r):
        rbase = jnp.full((L,), r * F, jnp.int32)
        xv0 = plsc.load_gather(x_v, [rbase + f0])
        xv1 = plsc.load_gather(x_v, [rbase + f1])
        return f0 * C + xv0, f1 * C + xv1

    def paint(buf, r, val):
        i0, i1 = row_targets(r)
        plsc.store_scatter(buf, [i0], val)
        plsc.store_scatter(buf, [i1], val, mask=m1)

    bufs = (buf0, buf1, buf2, buf3)
    sems = (sem0, sem1, sem2, sem3)
    NB = len(bufs)

    def zbody(k, _):
        o = k * (ZCHUNK * L)
        for j in range(ZCHUNK):
            for buf in bufs:
                buf[pl.ds(o + j * L, L)] = zeros
        return 0
    lax.fori_loop(0, OUT_W // (ZCHUNK * L), zbody, 0)

    for b in range(NB):
        paint(bufs[b], b, ones)
        pltpu.async_copy(bufs[b], out_hbm.at[base + b], sems[b])

    def body(j, _):
        for b in range(NB):
            r = NB * j + b
            pltpu.make_async_copy(bufs[b], out_hbm.at[base + r - NB],
                                  sems[b]).wait()
            paint(bufs[b], r - NB, zeros)
            paint(bufs[b], r, ones)
            pltpu.async_copy(bufs[b], out_hbm.at[base + r], sems[b])
        return 0
    lax.fori_loop(1, ROWS // NB, body, 0)

    for b in range(NB):
        pltpu.make_async_copy(bufs[b], out_hbm.at[base + ROWS - NB + b],
                              sems[b]).wait()


_sc_one_hot = functools.partial(
    pl.kernel,
    out_type=jax.ShapeDtypeStruct((B, OUT_W), jnp.float32),
    mesh=plsc.VectorSubcoreMesh(core_axis_name="c", subcore_axis_name="s"),
    compiler_params=pltpu.CompilerParams(needs_layout_passes=False),
    scratch_types=[
        pltpu.VMEM((ROWS * F,), jnp.int32),
        pltpu.VMEM((OUT_W,), jnp.float32),
        pltpu.VMEM((OUT_W,), jnp.float32),
        pltpu.VMEM((OUT_W,), jnp.float32),
        pltpu.VMEM((OUT_W,), jnp.float32),
        pltpu.SemaphoreType.DMA,
        pltpu.SemaphoreType.DMA,
        pltpu.SemaphoreType.DMA,
        pltpu.SemaphoreType.DMA,
    ],
)(_sc_one_hot_body)


@jax.jit
def kernel(x):
    return _sc_one_hot(x.astype(jnp.int32).reshape(B * F))


# speedup vs baseline: 3.1029x; 3.1029x over previous
import functools

import jax
import jax.numpy as jnp
from jax import lax
from jax.experimental import pallas as pl
from jax.experimental.pallas import tpu as pltpu
from jax.experimental.pallas import tpu_sc as plsc

B = 4096
F = 26
C = 1000
OUT_W = F * C
NC, NS, L = 2, 16, 16
NW = NC * NS
COLS = B // NW
CROWS = C // 5
KV = COLS // L


def _sc_one_hot_t_body(x_hbm, out_hbm, x_v, tile, sem):
    wid = lax.axis_index("s") * NC + lax.axis_index("c")
    col0 = wid * COLS

    pltpu.sync_copy(x_hbm.at[pl.ds(col0 * F, COLS * F)], x_v)

    lanes = lax.iota(jnp.int32, L)
    ones = jnp.full((L,), 1.0, jnp.float32)
    zeros = jnp.zeros((L,), jnp.float32)

    def zbody(r, _):
        for k in range(KV):
            tile[r, pl.ds(k * L, L)] = zeros
        return 0
    lax.fori_loop(0, CROWS, zbody, 0)

    def sweep(f, c0, val):
        for k in range(KV):
            bl = k * L + lanes
            xv = plsc.load_gather(x_v, [bl * F + f])
            rel = xv - c0
            m = (rel >= 0) & (rel < CROWS)
            rel = jnp.minimum(jnp.maximum(rel, 0), CROWS - 1)
            plsc.store_scatter(tile, [rel, bl], val, mask=m)

    def fbody(f, _):
        for h in range(C // CROWS):
            c0 = h * CROWS
            sweep(f, c0, ones)
            pltpu.async_copy(
                tile,
                out_hbm.at[pl.ds(f * C + c0, CROWS), pl.ds(col0, COLS)],
                sem).wait()
            sweep(f, c0, zeros)
        return 0
    lax.fori_loop(0, F, fbody, 0)


_sc_one_hot_t = functools.partial(
    pl.kernel,
    out_type=jax.ShapeDtypeStruct((OUT_W, B), jnp.float32),
    mesh=plsc.VectorSubcoreMesh(core_axis_name="c", subcore_axis_name="s"),
    compiler_params=pltpu.CompilerParams(needs_layout_passes=False),
    scratch_types=[
        pltpu.VMEM((COLS * F,), jnp.int32),
        pltpu.VMEM((CROWS, COLS), jnp.float32),
        pltpu.SemaphoreType.DMA,
    ],
)(_sc_one_hot_t_body)


@jax.jit
def kernel(x):
    out_t = _sc_one_hot_t(x.astype(jnp.int32).reshape(B * F))
    return out_t.T


# speedup vs baseline: 3.4290x; 1.1051x over previous
import functools

import jax
import jax.numpy as jnp
from jax import lax
from jax.experimental import pallas as pl
from jax.experimental.pallas import tpu as pltpu
from jax.experimental.pallas import tpu_sc as plsc

B = 4096
F = 26
C = 1000
OUT_W = F * C
NC, NS, L = 2, 16, 16
NW = NC * NS
COLS = B // NW
CROWS = C // 5
CH = C // CROWS
ITEMS = F * CH
KV = COLS // L


def _sc_one_hot_t_body(x_hbm, out_hbm, x_v, tile0, tile1, sem0, sem1):
    wid = lax.axis_index("s") * NC + lax.axis_index("c")
    col0 = wid * COLS

    pltpu.sync_copy(x_hbm.at[pl.ds(col0 * F, COLS * F)], x_v)

    lanes = lax.iota(jnp.int32, L)
    ones = jnp.full((L,), 1.0, jnp.float32)
    zeros = jnp.zeros((L,), jnp.float32)
    tiles = (tile0, tile1)
    sems = (sem0, sem1)

    def zbody(r, _):
        for k in range(KV):
            tile0[r, pl.ds(k * L, L)] = zeros
            tile1[r, pl.ds(k * L, L)] = zeros
        return 0
    lax.fori_loop(0, CROWS, zbody, 0)

    def item_fc(i):
        f = i // CH
        c0 = (i - CH * f) * CROWS
        return f, c0

    def dst(f, c0):
        return out_hbm.at[pl.ds(f * C + c0, CROWS), pl.ds(col0, COLS)]

    def sweep(tile, f, c0, val):
        for k in range(KV):
            bl = k * L + lanes
            xv = plsc.load_gather(x_v, [bl * F + f])
            rel = xv - c0
            m = (rel >= 0) & (rel < CROWS)
            rel = jnp.minimum(jnp.maximum(rel, 0), CROWS - 1)
            plsc.store_scatter(tile, [rel, bl], val, mask=m)

    def paint_start(b, i):
        f, c0 = item_fc(i)
        sweep(tiles[b], f, c0, ones)
        pltpu.async_copy(tiles[b], dst(f, c0), sems[b])

    for b in range(2):
        paint_start(b, b)

    def body(j, _):
        for b in range(2):
            i = 2 * j + b
            f2, c02 = item_fc(i - 2)
            pltpu.make_async_copy(tiles[b], dst(f2, c02), sems[b]).wait()
            sweep(tiles[b], f2, c02, zeros)
            paint_start(b, i)
        return 0
    lax.fori_loop(1, ITEMS // 2, body, 0)

    for b in range(2):
        f2, c02 = item_fc(ITEMS - 2 + b)
        pltpu.make_async_copy(tiles[b], dst(f2, c02), sems[b]).wait()


_sc_one_hot_t = functools.partial(
    pl.kernel,
    out_type=jax.ShapeDtypeStruct((OUT_W, B), jnp.float32),
    mesh=plsc.VectorSubcoreMesh(core_axis_name="c", subcore_axis_name="s"),
    compiler_params=pltpu.CompilerParams(needs_layout_passes=False),
    scratch_types=[
        pltpu.VMEM((COLS * F,), jnp.int32),
        pltpu.VMEM((CROWS, COLS), jnp.float32),
        pltpu.VMEM((CROWS, COLS), jnp.float32),
        pltpu.SemaphoreType.DMA,
        pltpu.SemaphoreType.DMA,
    ],
)(_sc_one_hot_t_body)


@jax.jit
def kernel(x):
    out_t = _sc_one_hot_t(x.astype(jnp.int32).reshape(B * F))
    return out_t.T


# speedup vs baseline: 3.4529x; 1.0070x over previous
import functools

import jax
import jax.numpy as jnp
from jax import lax
from jax.experimental import pallas as pl
from jax.experimental.pallas import tpu as pltpu
from jax.experimental.pallas import tpu_sc as plsc

B = 4096
F = 26
C = 1000
OUT_W = F * C
NC, NS, L = 2, 16, 16
NW = NC * NS
COLS = B // NW
CROWS = C // 5
CH = C // CROWS
ITEMS = F * CH
KV = COLS // L


def _sc_one_hot_t_body(xt_hbm, out_hbm, x_v, tile0, tile1, sem0, sem1):
    wid = lax.axis_index("s") * NC + lax.axis_index("c")
    col0 = wid * COLS

    pltpu.sync_copy(xt_hbm.at[:, pl.ds(col0, COLS)], x_v)

    lanes = lax.iota(jnp.int32, L)
    ones = jnp.full((L,), 1.0, jnp.float32)
    zeros = jnp.zeros((L,), jnp.float32)
    tiles = (tile0, tile1)
    sems = (sem0, sem1)

    def zbody(r, _):
        for k in range(KV):
            tile0[r, pl.ds(k * L, L)] = zeros
            tile1[r, pl.ds(k * L, L)] = zeros
        return 0
    lax.fori_loop(0, CROWS, zbody, 0)

    def item_fc(i):
        f = i // CH
        c0 = (i - CH * f) * CROWS
        return f, c0

    def dst(f, c0):
        return out_hbm.at[pl.ds(f * C + c0, CROWS), pl.ds(col0, COLS)]

    def sweep(tile, f, c0, val):
        for k in range(KV):
            bl = k * L + lanes
            xv = x_v[f, pl.ds(k * L, L)]
            rel = xv - c0
            m = (rel >= 0) & (rel < CROWS)
            rel = jnp.minimum(jnp.maximum(rel, 0), CROWS - 1)
            plsc.store_scatter(tile, [rel, bl], val, mask=m)

    def paint_start(b, i):
        f, c0 = item_fc(i)
        sweep(tiles[b], f, c0, ones)
        pltpu.async_copy(tiles[b], dst(f, c0), sems[b])

    for b in range(2):
        paint_start(b, b)

    def body(j, _):
        for b in range(2):
            i = 2 * j + b
            f2, c02 = item_fc(i - 2)
            pltpu.make_async_copy(tiles[b], dst(f2, c02), sems[b]).wait()
            sweep(tiles[b], f2, c02, zeros)
            paint_start(b, i)
        return 0
    lax.fori_loop(1, ITEMS // 2, body, 0)

    for b in range(2):
        f2, c02 = item_fc(ITEMS - 2 + b)
        pltpu.make_async_copy(tiles[b], dst(f2, c02), sems[b]).wait()


_sc_one_hot_t = functools.partial(
    pl.kernel,
    out_type=jax.ShapeDtypeStruct((OUT_W, B), jnp.float32),
    mesh=plsc.VectorSubcoreMesh(core_axis_name="c", subcore_axis_name="s"),
    compiler_params=pltpu.CompilerParams(needs_layout_passes=False),
    scratch_types=[
        pltpu.VMEM((F, COLS), jnp.int32),
        pltpu.VMEM((CROWS, COLS), jnp.float32),
        pltpu.VMEM((CROWS, COLS), jnp.float32),
        pltpu.SemaphoreType.DMA,
        pltpu.SemaphoreType.DMA,
    ],
)(_sc_one_hot_t_body)


@jax.jit
def kernel(x):
    out_t = _sc_one_hot_t(x.astype(jnp.int32).T)
    return out_t.T
